# Initial kernel scaffold; baseline (speedup 1.0000x reference)
import jax
import jax.numpy as jnp
from jax.experimental import pallas as pl


def kernel(features, edge_index, W0, b0, W1, b1, Wd):
    raise NotImplementedError("write your pallas kernel here")


# speedup vs baseline: 5.6075x; 5.6075x over previous
import functools

import jax
import jax.numpy as jnp
from jax import lax
from jax.experimental import pallas as pl
from jax.experimental.pallas import tpu as pltpu
from jax.experimental.pallas import tpu_sc as plsc

N = 10000
E = 320000
H = 128
NPAD = 10240
NC = 2
NSC = 16
L = 16

EPT = E // NSC
CH = 128
NFULL = EPT // CH
REM = EPT - NFULL * CH
ROWS_PT = NPAD // NSC
WCH = 80


def _norm(d):
    return jnp.where(d > 0.0, lax.rsqrt(jnp.maximum(d, 1.0)), 0.0)


@functools.lru_cache(maxsize=None)
def _make_sc_stats():
    mesh = plsc.VectorSubcoreMesh(core_axis_name="c", subcore_axis_name="s")

    @functools.partial(
        pl.kernel,
        out_type=(
            jax.ShapeDtypeStruct((NPAD, H), jnp.float32),
            jax.ShapeDtypeStruct((NPAD, H), jnp.float32),
            jax.ShapeDtypeStruct((N, H), jnp.float32),
        ),
        mesh=mesh,
        compiler_params=pltpu.CompilerParams(needs_layout_passes=False),
        scratch_types=(
            pltpu.VMEM((640,), jnp.float32),
            pltpu.VMEM((CH,), jnp.float32),
            pltpu.VMEM((CH,), jnp.int32),
            pltpu.VMEM((REM,), jnp.int32),
            pltpu.VMEM((104,), jnp.int32),
            pltpu.VMEM((104, H), jnp.float32),
            pltpu.VMEM((16,), jnp.int32),
            pltpu.VMEM((16, H), jnp.float32),
            pltpu.VMEM((ROWS_PT,), jnp.float32),
            pltpu.VMEM((64, H), jnp.float32),
            pltpu.VMEM_SHARED((NPAD,), jnp.float32),
            pltpu.SemaphoreType.DMA,
        ),
    )
    def sc_stats(ecat, feat, permv, nsb, ndb, pout,
                 zb, ob, ib, ib32, pidx, prow, pidx16, prow16,
                 dbuf, rowbuf, dacc, sem):
        c = lax.axis_index("c")
        s = lax.axis_index("s")

        for j in range(ROWS_PT // L):
            zb[pl.ds(j * L, L)] = jnp.zeros((L,), jnp.float32)
        for j in range(CH // L):
            ob[pl.ds(j * L, L)] = jnp.ones((L,), jnp.float32)
        pltpu.sync_copy(zb, dacc.at[pl.ds(s * ROWS_PT, ROWS_PT)])
        plsc.subcore_barrier()

        base = c * E + s * EPT

        def dstep(k, _):
            off = base + k * CH
            pltpu.sync_copy(ecat.at[pl.ds(off, CH)], ib)
            pltpu.sync_copy(ob, dacc.at[ib], add=True)
            return ()

        lax.fori_loop(0, NFULL, dstep, ())
        pltpu.sync_copy(ecat.at[pl.ds(base + NFULL * CH, REM)], ib32)
        pltpu.sync_copy(ob.at[pl.ds(0, REM)], dacc.at[ib32], add=True)

        w = s * NC + c
        for k in range(3):
            pb = w * 312 + k * 104
            pltpu.sync_copy(permv.at[pl.ds(pb, 104)], pidx)
            pltpu.async_copy(feat.at[pidx], prow, sem).wait()
            pltpu.sync_copy(prow, pout.at[pl.ds(pb, 104)])

        @pl.when(w == NC * NSC - 1)
        def _tail():
            pltpu.sync_copy(permv.at[pl.ds(9984, 16)], pidx16)
            pltpu.async_copy(feat.at[pidx16], prow16, sem).wait()
            pltpu.sync_copy(prow16, pout.at[pl.ds(9984, 16)])

        plsc.subcore_barrier()

        pltpu.sync_copy(dacc.at[pl.ds(s * ROWS_PT, ROWS_PT)], dbuf)

        def bcast(out_ref):
            def chunk(ch, _):
                for r in range(64):
                    nv = plsc.load_gather(
                        dbuf, [jnp.full((L,), ch * 64 + r, jnp.int32)])
                    for j in range(H // L):
                        rowbuf[r, pl.ds(j * L, L)] = nv
                pltpu.sync_copy(
                    rowbuf, out_ref.at[pl.ds(s * ROWS_PT + ch * 64, 64)])
                return ()
            lax.fori_loop(0, ROWS_PT // 64, chunk, ())

        @pl.when(c == 0)
        def _w0():
            bcast(nsb)

        @pl.when(c == 1)
        def _w1():
            bcast(ndb)

    return sc_stats


@functools.lru_cache(maxsize=None)
def _make_sc_spmm():
    mesh = plsc.VectorSubcoreMesh(core_axis_name="c", subcore_axis_name="s")

    @functools.partial(
        pl.kernel,
        out_type=jax.ShapeDtypeStruct((2 * N, H), jnp.float32),
        mesh=mesh,
        compiler_params=pltpu.CompilerParams(needs_layout_passes=False),
        scratch_types=(
            pltpu.VMEM((CH,), jnp.int32),
            pltpu.VMEM((CH,), jnp.int32),
            pltpu.VMEM((CH,), jnp.int32),
            pltpu.VMEM((CH,), jnp.int32),
            pltpu.VMEM((CH, H), jnp.float32),
            pltpu.VMEM((CH, H), jnp.float32),
            pltpu.VMEM((REM,), jnp.int32),
            pltpu.VMEM((REM,), jnp.int32),
            pltpu.VMEM((REM, H), jnp.float32),
            pltpu.VMEM((WCH, H), jnp.float32),
            pltpu.VMEM_SHARED((NPAD, H), jnp.float32),
            pltpu.SemaphoreType.DMA,
            pltpu.SemaphoreType.DMA,
            pltpu.SemaphoreType.DMA,
            pltpu.SemaphoreType.DMA,
            pltpu.SemaphoreType.DMA,
            pltpu.SemaphoreType.DMA,
        ),
    )
    def sc_spmm(tbl, ecat, agg,
                srcb0, dstb0, srcb1, dstb1, rows0, rows1,
                rsrc, rdst, rrows, stg, acc,
                ss0, sd0, ss1, sd1, sg0, sg1):
        c = lax.axis_index("c")
        s = lax.axis_index("s")
        roff = c * N

        def zrow(r, _):
            for j in range(H // L):
                stg[r, pl.ds(j * L, L)] = jnp.zeros((L,), jnp.float32)
            return ()
        lax.fori_loop(0, WCH, zrow, ())

        def zchunk(k, _):
            pltpu.sync_copy(stg, acc.at[pl.ds(s * ROWS_PT + k * WCH, WCH)])
            return ()
        lax.fori_loop(0, ROWS_PT // WCH, zchunk, ())
        plsc.subcore_barrier()

        base = s * EPT

        def adj(ref):
            for j in range(CH // L):
                ref[pl.ds(j * L, L)] = ref[pl.ds(j * L, L)] + roff

        pltpu.async_copy(ecat.at[pl.ds(base, CH)], srcb0, ss0)
        pltpu.async_copy(ecat.at[pl.ds(E + base, CH)], dstb0, sd0)
        pltpu.async_copy(ecat.at[pl.ds(base + CH, CH)], srcb1, ss1)
        pltpu.async_copy(ecat.at[pl.ds(E + base + CH, CH)], dstb1, sd1)

        def body(i, _):
            off0 = base + (2 * i) * CH
            off1 = off0 + CH
            live = off0 + 2 * CH < base + NFULL * CH
            offn0 = jnp.where(live, off0 + 2 * CH, base)
            offn1 = jnp.where(live, off1 + 2 * CH, base)

            pltpu.make_async_copy(
                ecat.at[pl.ds(off0, CH)], srcb0, ss0).wait()
            pltpu.make_async_copy(
                ecat.at[pl.ds(E + off0, CH)], dstb0, sd0).wait()
            adj(srcb0)
            g0 = pltpu.async_copy(tbl.at[srcb0], rows0, sg0)

            pltpu.make_async_copy(
                ecat.at[pl.ds(off1, CH)], srcb1, ss1).wait()
            pltpu.make_async_copy(
                ecat.at[pl.ds(E + off1, CH)], dstb1, sd1).wait()
            adj(srcb1)
            g0.wait()
            g1 = pltpu.async_copy(tbl.at[srcb1], rows1, sg1)
            pltpu.sync_copy(rows0, acc.at[dstb0], add=True)
            pltpu.async_copy(ecat.at[pl.ds(offn0, CH)], srcb0, ss0)
            pltpu.async_copy(ecat.at[pl.ds(E + offn0, CH)], dstb0, sd0)
            g1.wait()
            pltpu.sync_copy(rows1, acc.at[dstb1], add=True)
            pltpu.async_copy(ecat.at[pl.ds(offn1, CH)], srcb1, ss1)
            pltpu.async_copy(ecat.at[pl.ds(E + offn1, CH)], dstb1, sd1)
            return ()

        lax.fori_loop(0, NFULL // 2, body, ())
        pltpu.make_async_copy(ecat.at[pl.ds(base, CH)], srcb0, ss0).wait()
        pltpu.make_async_copy(ecat.at[pl.ds(base, CH)], dstb0, sd0).wait()
        pltpu.make_async_copy(ecat.at[pl.ds(base, CH)], srcb1, ss1).wait()
        pltpu.make_async_copy(ecat.at[pl.ds(base, CH)], dstb1, sd1).wait()

        rbase = base + NFULL * CH
        pltpu.sync_copy(ecat.at[pl.ds(rbase, REM)], rsrc)
        pltpu.sync_copy(ecat.at[pl.ds(E + rbase, REM)], rdst)
        for j in range(REM // L):
            rsrc[pl.ds(j * L, L)] = rsrc[pl.ds(j * L, L)] + roff
        pltpu.async_copy(tbl.at[rsrc], rrows, sg0).wait()
        pltpu.sync_copy(rrows, acc.at[rdst], add=True)

        plsc.subcore_barrier()

        nch = jnp.where(s == NSC - 1, (N - (NSC - 1) * ROWS_PT) // WCH,
                        ROWS_PT // WCH)

        def wchunk(k, _):
            r0 = s * ROWS_PT + k * WCH
            pltpu.sync_copy(acc.at[pl.ds(r0, WCH)], stg)
            pltpu.sync_copy(stg, agg.at[pl.ds(roff + r0, WCH)])
            return ()
        lax.fori_loop(0, nch, wchunk, ())

    return sc_spmm


def _tc1_body(f_ref, p_ref, ns_ref, o_ref):
    h = pl.program_id(0)

    @pl.when(h == 0)
    def _pos():
        o_ref[...] = f_ref[...] * _norm(ns_ref[...])

    @pl.when(h == 1)
    def _neg():
        o_ref[...] = p_ref[...] * _norm(ns_ref[...])


def _tc2_body(a_ref, nd_ref, ns_ref, w_ref, b_ref, o_ref):
    z = a_ref[...] * _norm(nd_ref[...])
    y = jnp.dot(z, w_ref[...], preferred_element_type=jnp.float32)
    y = jnp.maximum(y + b_ref[0:1, :], 0.0)
    o_ref[...] = y * _norm(ns_ref[...])


def _tca_body(a_ref, nd_ref, o_ref):
    @pl.when(pl.program_id(0) == 0)
    def _init():
        o_ref[...] = jnp.zeros_like(o_ref)

    cs = jnp.sum(a_ref[...] * _norm(nd_ref[...]), axis=0, keepdims=True)
    o_ref[...] += jnp.broadcast_to(cs, o_ref.shape)


def _tcb_body(cs_ref, w1_ref, b1_ref, wd_ref, s2_ref, c0_ref):
    m = cs_ref[0:1, :] / jnp.float32(N)
    summary = jax.nn.sigmoid(
        jnp.dot(m, w1_ref[...], preferred_element_type=jnp.float32)
        + b1_ref[0:1, :])
    s = lax.dot_general(summary, wd_ref[...], (((1,), (1,)), ((), ())),
                        preferred_element_type=jnp.float32)
    s2 = lax.dot_general(s, w1_ref[...], (((1,), (1,)), ((), ())),
                         preferred_element_type=jnp.float32)
    s2_ref[...] = jnp.broadcast_to(s2, s2_ref.shape)
    c0 = jnp.sum(b1_ref[0:1, :] * s)
    c0_ref[...] = jnp.full(c0_ref.shape, c0, jnp.float32)


def _tcc_body(a_ref, nd_ref, s2_ref, c0_ref, o_ref):
    h = pl.program_id(0)

    @pl.when(jnp.logical_and(h == 0, pl.program_id(1) == 0))
    def _init():
        o_ref[...] = jnp.zeros_like(o_ref)

    z = a_ref[...] * _norm(nd_ref[...])
    lg = jnp.sum(z * s2_ref[0:1, :], axis=1) + c0_ref[0, 0]
    x = jnp.where(h == 0, -lg, lg)
    v = jnp.maximum(x, 0.0) + jnp.log(1.0 + jnp.exp(-jnp.abs(x)))
    o_ref[...] += jnp.full(o_ref.shape, jnp.sum(v) / jnp.float32(N))


_BR = 1000
_f32 = jnp.float32


def _rows_spec(nblk_offset=False):
    if nblk_offset:
        return pl.BlockSpec((_BR, H), lambda h, r: (h * (N // _BR) + r, 0))
    return pl.BlockSpec((_BR, H), lambda h, r: (r, 0))


def kernel(features, edge_index, W0, b0, W1, b1, Wd):
    ecat = edge_index.reshape(-1)
    perm = jax.random.permutation(jax.random.key(42), N).astype(jnp.int32)
    b0r = jnp.broadcast_to(b0.reshape(1, H), (8, H))
    b1r = jnp.broadcast_to(b1.reshape(1, H), (8, H))

    nsb, ndb, P = _make_sc_stats()(ecat, features, perm)

    xcat = pl.pallas_call(
        _tc1_body,
        grid=(2, N // _BR),
        in_specs=[_rows_spec(), _rows_spec(), _rows_spec()],
        out_specs=_rows_spec(nblk_offset=True),
        out_shape=jax.ShapeDtypeStruct((2 * N, H), _f32),
    )(features, P, nsb)

    agg1 = _make_sc_spmm()(xcat, ecat)

    h1 = pl.pallas_call(
        _tc2_body,
        grid=(2, N // _BR),
        in_specs=[
            _rows_spec(nblk_offset=True),
            _rows_spec(),
            _rows_spec(),
            pl.BlockSpec((H, H), lambda h, r: (0, 0)),
            pl.BlockSpec((8, H), lambda h, r: (0, 0)),
        ],
        out_specs=_rows_spec(nblk_offset=True),
        out_shape=jax.ShapeDtypeStruct((2 * N, H), _f32),
    )(agg1, ndb, nsb, W0, b0r)

    agg2 = _make_sc_spmm()(h1, ecat)

    cs = pl.pallas_call(
        _tca_body,
        grid=(N // _BR,),
        in_specs=[pl.BlockSpec((_BR, H), lambda r: (r, 0)),
                  pl.BlockSpec((_BR, H), lambda r: (r, 0))],
        out_specs=pl.BlockSpec((8, H), lambda r: (0, 0)),
        out_shape=jax.ShapeDtypeStruct((8, H), _f32),
    )(agg2, ndb)

    s2, c0 = pl.pallas_call(
        _tcb_body,
        out_shape=(jax.ShapeDtypeStruct((8, H), _f32),
                   jax.ShapeDtypeStruct((8, 8), _f32)),
    )(cs, W1, b1r, Wd)

    loss = pl.pallas_call(
        _tcc_body,
        grid=(2, N // _BR),
        in_specs=[
            _rows_spec(nblk_offset=True),
            _rows_spec(),
            pl.BlockSpec((8, H), lambda h, r: (0, 0)),
            pl.BlockSpec((8, 8), lambda h, r: (0, 0)),
        ],
        out_specs=pl.BlockSpec((8, 8), lambda h, r: (0, 0)),
        out_shape=jax.ShapeDtypeStruct((8, 8), _f32),
    )(agg2, ndb, s2, c0)

    return loss[0, 0]


# speedup vs baseline: 7.7604x; 1.3839x over previous
import functools

import jax
import jax.numpy as jnp
from jax import lax
from jax.experimental import pallas as pl
from jax.experimental.pallas import tpu as pltpu
from jax.experimental.pallas import tpu_sc as plsc

N = 10000
E = 320000
H = 128
NPAD = 10240
NC = 2
NSC = 16
L = 16

EPT = E // NSC
CH = 80
NFULL = EPT // CH
NB4 = NFULL // 4
LEFT = NFULL - NB4 * 4
ROWS_PT = NPAD // NSC
WCH = 80


def _norm(d):
    return jnp.where(d > 0.0, lax.rsqrt(jnp.maximum(d, 1.0)), 0.0)


@functools.lru_cache(maxsize=None)
def _make_sc_stats():
    mesh = plsc.VectorSubcoreMesh(core_axis_name="c", subcore_axis_name="s")

    @functools.partial(
        pl.kernel,
        out_type=(
            jax.ShapeDtypeStruct((NPAD, H), jnp.float32),
            jax.ShapeDtypeStruct((NPAD, H), jnp.float32),
            jax.ShapeDtypeStruct((N, H), jnp.float32),
        ),
        mesh=mesh,
        compiler_params=pltpu.CompilerParams(needs_layout_passes=False),
        scratch_types=(
            pltpu.VMEM((640,), jnp.float32),
            pltpu.VMEM((CH,), jnp.float32),
            tuple(pltpu.VMEM((CH,), jnp.int32) for _ in range(4)),
            tuple(pltpu.VMEM((CH,), jnp.int32) for _ in range(4)),
            tuple(pltpu.SemaphoreType.DMA for _ in range(4)),
            tuple(pltpu.SemaphoreType.DMA for _ in range(4)),
            pltpu.VMEM((104,), jnp.int32),
            pltpu.VMEM((104, H), jnp.float32),
            pltpu.VMEM((16,), jnp.int32),
            pltpu.VMEM((16, H), jnp.float32),
            pltpu.VMEM((ROWS_PT,), jnp.float32),
            pltpu.VMEM((64, H), jnp.float32),
            pltpu.VMEM_SHARED((NPAD,), jnp.float32),
            pltpu.SemaphoreType.DMA,
        ),
    )
    def sc_stats(ecat, feat, permv, nsb, ndb, pout,
                 zb, ob, ib, ib2, dis, dcs, pidx, prow, pidx16,
                 prow16, dbuf, rowbuf, dacc, sem):
        c = lax.axis_index("c")
        s = lax.axis_index("s")

        for j in range(ROWS_PT // L):
            zb[pl.ds(j * L, L)] = jnp.zeros((L,), jnp.float32)
        for j in range(CH // L):
            ob[pl.ds(j * L, L)] = jnp.ones((L,), jnp.float32)
        pltpu.sync_copy(zb, dacc.at[pl.ds(s * ROWS_PT, ROWS_PT)])
        plsc.subcore_barrier()

        base = c * E + s * EPT
        for u in range(4):
            pltpu.async_copy(ecat.at[pl.ds(base + u * CH, CH)],
                             ib[u], dis[u])

        def dstep(i, _):
            for u in range(4):
                k = 4 * i + u
                off = base + k * CH
                pltpu.make_async_copy(
                    ecat.at[pl.ds(off, CH)], ib[u], dis[u]).wait()

                @pl.when(i > 0)
                def _w():
                    pltpu.make_async_copy(
                        ob, dacc.at[ib2[u]], dcs[u]).wait()
                for j in range(CH // L):
                    ib2[u][pl.ds(j * L, L)] = ib[u][pl.ds(j * L, L)]
                kn = jnp.where(k + 4 < NFULL, k + 4, 0)
                pltpu.async_copy(ecat.at[pl.ds(base + kn * CH, CH)],
                                 ib[u], dis[u])
                pltpu.async_copy(ob, dacc.at[ib2[u]], dcs[u], add=True)
            return ()

        lax.fori_loop(0, NB4, dstep, ())
        for u in range(4):
            pltpu.make_async_copy(
                ecat.at[pl.ds(base, CH)], ib[u], dis[u]).wait()
        for u in range(LEFT):
            pltpu.make_async_copy(ob, dacc.at[ib2[u]], dcs[u]).wait()
            for j in range(CH // L):
                ib2[u][pl.ds(j * L, L)] = ib[u][pl.ds(j * L, L)]
            pltpu.async_copy(ob, dacc.at[ib2[u]], dcs[u], add=True)
        for u in range(4):
            pltpu.make_async_copy(ob, dacc.at[ib2[u]], dcs[u]).wait()

        w = s * NC + c
        for k in range(3):
            pb = w * 312 + k * 104
            pltpu.sync_copy(permv.at[pl.ds(pb, 104)], pidx)
            pltpu.async_copy(feat.at[pidx], prow, sem).wait()
            pltpu.sync_copy(prow, pout.at[pl.ds(pb, 104)])

        @pl.when(w == NC * NSC - 1)
        def _tail():
            pltpu.sync_copy(permv.at[pl.ds(9984, 16)], pidx16)
            pltpu.async_copy(feat.at[pidx16], prow16, sem).wait()
            pltpu.sync_copy(prow16, pout.at[pl.ds(9984, 16)])

        plsc.subcore_barrier()

        pltpu.sync_copy(dacc.at[pl.ds(s * ROWS_PT, ROWS_PT)], dbuf)

        def bcast(out_ref):
            def chunk(ch, _):
                for r in range(64):
                    nv = plsc.load_gather(
                        dbuf, [jnp.full((L,), ch * 64 + r, jnp.int32)])
                    for j in range(H // L):
                        rowbuf[r, pl.ds(j * L, L)] = nv
                pltpu.sync_copy(
                    rowbuf, out_ref.at[pl.ds(s * ROWS_PT + ch * 64, 64)])
                return ()
            lax.fori_loop(0, ROWS_PT // 64, chunk, ())

        @pl.when(c == 0)
        def _w0():
            bcast(nsb)

        @pl.when(c == 1)
        def _w1():
            bcast(ndb)

    return sc_stats


@functools.lru_cache(maxsize=None)
def _make_sc_spmm():
    mesh = plsc.VectorSubcoreMesh(core_axis_name="c", subcore_axis_name="s")

    @functools.partial(
        pl.kernel,
        out_type=jax.ShapeDtypeStruct((2 * N, H), jnp.float32),
        mesh=mesh,
        compiler_params=pltpu.CompilerParams(needs_layout_passes=False),
        scratch_types=(
            tuple(pltpu.VMEM((CH,), jnp.int32) for _ in range(4)),
            tuple(pltpu.VMEM((CH,), jnp.int32) for _ in range(4)),
            tuple(pltpu.VMEM((CH,), jnp.int32) for _ in range(4)),
            tuple(pltpu.VMEM((CH,), jnp.int32) for _ in range(4)),
            tuple(pltpu.VMEM((CH, H), jnp.float32) for _ in range(4)),
            pltpu.VMEM_SHARED((NPAD, H), jnp.float32),
            tuple(pltpu.SemaphoreType.DMA for _ in range(4)),
            tuple(pltpu.SemaphoreType.DMA for _ in range(4)),
            tuple(pltpu.SemaphoreType.DMA for _ in range(4)),
            tuple(pltpu.SemaphoreType.DMA for _ in range(4)),
        ),
    )
    def sc_spmm(tbl, ecat, agg,
                srcb, dstb, srcb2, dstb2, rows, acc,
                sis, sid, sg, sc):
        c = lax.axis_index("c")
        s = lax.axis_index("s")
        roff = c * N

        def zrow(r, _):
            for j in range(H // L):
                rows[0][r, pl.ds(j * L, L)] = jnp.zeros((L,), jnp.float32)
            return ()
        lax.fori_loop(0, WCH, zrow, ())

        def zchunk(k, _):
            pltpu.sync_copy(rows[0],
                            acc.at[pl.ds(s * ROWS_PT + k * WCH, WCH)])
            return ()
        lax.fori_loop(0, ROWS_PT // WCH, zchunk, ())
        plsc.subcore_barrier()

        base = s * EPT

        for u in range(4):
            pltpu.async_copy(ecat.at[pl.ds(base + u * CH, CH)],
                             srcb[u], sis[u])
            pltpu.async_copy(ecat.at[pl.ds(E + base + u * CH, CH)],
                             dstb[u], sid[u])

        def body(i, _):
            for u in range(4):
                k = 4 * i + u
                off = base + k * CH
                pltpu.make_async_copy(
                    ecat.at[pl.ds(off, CH)], srcb[u], sis[u]).wait()
                pltpu.make_async_copy(
                    ecat.at[pl.ds(E + off, CH)], dstb[u], sid[u]).wait()

                @pl.when(i > 0)
                def _w():
                    pltpu.make_async_copy(
                        rows[u], acc.at[dstb2[u]], sc[u]).wait()
                for j in range(CH // L):
                    srcb2[u][pl.ds(j * L, L)] = (
                        srcb[u][pl.ds(j * L, L)] + roff)
                    dstb2[u][pl.ds(j * L, L)] = dstb[u][pl.ds(j * L, L)]
                kn = jnp.where(k + 4 < NFULL, k + 4, 0)
                pltpu.async_copy(ecat.at[pl.ds(base + kn * CH, CH)],
                                 srcb[u], sis[u])
                pltpu.async_copy(ecat.at[pl.ds(E + base + kn * CH, CH)],
                                 dstb[u], sid[u])
                pltpu.async_copy(tbl.at[srcb2[u]], rows[u], sg[u])
            for u in range(4):
                pltpu.make_async_copy(tbl.at[srcb2[u]], rows[u],
                                      sg[u]).wait()
                pltpu.async_copy(rows[u], acc.at[dstb2[u]], sc[u],
                                 add=True)
            return ()

        lax.fori_loop(0, NB4, body, ())
        for u in range(4):
            pltpu.make_async_copy(
                ecat.at[pl.ds(base, CH)], srcb[u], sis[u]).wait()
            pltpu.make_async_copy(
                ecat.at[pl.ds(base, CH)], dstb[u], sid[u]).wait()
        for u in range(LEFT):
            pltpu.make_async_copy(rows[u], acc.at[dstb2[u]], sc[u]).wait()
            for j in range(CH // L):
                srcb2[u][pl.ds(j * L, L)] = srcb[u][pl.ds(j * L, L)] + roff
                dstb2[u][pl.ds(j * L, L)] = dstb[u][pl.ds(j * L, L)]
            pltpu.async_copy(tbl.at[srcb2[u]], rows[u], sg[u])
        for u in range(LEFT):
            pltpu.make_async_copy(tbl.at[srcb2[u]], rows[u], sg[u]).wait()
            pltpu.async_copy(rows[u], acc.at[dstb2[u]], sc[u], add=True)
        for u in range(LEFT, 4):
            pltpu.make_async_copy(rows[u], acc.at[dstb2[u]], sc[u]).wait()
        for u in range(LEFT):
            pltpu.make_async_copy(rows[u], acc.at[dstb2[u]], sc[u]).wait()

        plsc.subcore_barrier()

        nch = jnp.where(s == NSC - 1, (N - (NSC - 1) * ROWS_PT) // WCH,
                        ROWS_PT // WCH)

        def wchunk(k, _):
            r0 = s * ROWS_PT + k * WCH
            pltpu.sync_copy(acc.at[pl.ds(r0, WCH)], rows[0])
            pltpu.sync_copy(rows[0], agg.at[pl.ds(roff + r0, WCH)])
            return ()
        lax.fori_loop(0, nch, wchunk, ())

    return sc_spmm


def _tc1_body(f_ref, p_ref, ns_ref, o_ref):
    h = pl.program_id(0)

    @pl.when(h == 0)
    def _pos():
        o_ref[...] = f_ref[...] * _norm(ns_ref[...])

    @pl.when(h == 1)
    def _neg():
        o_ref[...] = p_ref[...] * _norm(ns_ref[...])


def _tc2_body(a_ref, nd_ref, ns_ref, w_ref, b_ref, o_ref):
    z = a_ref[...] * _norm(nd_ref[...])
    y = jnp.dot(z, w_ref[...], preferred_element_type=jnp.float32)
    y = jnp.maximum(y + b_ref[0:1, :], 0.0)
    o_ref[...] = y * _norm(ns_ref[...])


def _tca_body(a_ref, nd_ref, o_ref):
    @pl.when(pl.program_id(0) == 0)
    def _init():
        o_ref[...] = jnp.zeros_like(o_ref)

    cs = jnp.sum(a_ref[...] * _norm(nd_ref[...]), axis=0, keepdims=True)
    o_ref[...] += jnp.broadcast_to(cs, o_ref.shape)


def _tcb_body(cs_ref, w1_ref, b1_ref, wd_ref, s2_ref, c0_ref):
    m = cs_ref[0:1, :] / jnp.float32(N)
    summary = jax.nn.sigmoid(
        jnp.dot(m, w1_ref[...], preferred_element_type=jnp.float32)
        + b1_ref[0:1, :])
    s = lax.dot_general(summary, wd_ref[...], (((1,), (1,)), ((), ())),
                        preferred_element_type=jnp.float32)
    s2 = lax.dot_general(s, w1_ref[...], (((1,), (1,)), ((), ())),
                         preferred_element_type=jnp.float32)
    s2_ref[...] = jnp.broadcast_to(s2, s2_ref.shape)
    c0 = jnp.sum(b1_ref[0:1, :] * s)
    c0_ref[...] = jnp.full(c0_ref.shape, c0, jnp.float32)


def _tcc_body(a_ref, nd_ref, s2_ref, c0_ref, o_ref):
    h = pl.program_id(0)

    @pl.when(jnp.logical_and(h == 0, pl.program_id(1) == 0))
    def _init():
        o_ref[...] = jnp.zeros_like(o_ref)

    z = a_ref[...] * _norm(nd_ref[...])
    lg = jnp.sum(z * s2_ref[0:1, :], axis=1) + c0_ref[0, 0]
    x = jnp.where(h == 0, -lg, lg)
    v = jnp.maximum(x, 0.0) + jnp.log(1.0 + jnp.exp(-jnp.abs(x)))
    o_ref[...] += jnp.full(o_ref.shape, jnp.sum(v) / jnp.float32(N))


_BR = 1000
_f32 = jnp.float32


def _rows_spec(nblk_offset=False):
    if nblk_offset:
        return pl.BlockSpec((_BR, H), lambda h, r: (h * (N // _BR) + r, 0))
    return pl.BlockSpec((_BR, H), lambda h, r: (r, 0))


def kernel(features, edge_index, W0, b0, W1, b1, Wd):
    ecat = edge_index.reshape(-1)
    perm = jax.random.permutation(jax.random.key(42), N).astype(jnp.int32)
    b0r = jnp.broadcast_to(b0.reshape(1, H), (8, H))
    b1r = jnp.broadcast_to(b1.reshape(1, H), (8, H))

    nsb, ndb, P = _make_sc_stats()(ecat, features, perm)

    xcat = pl.pallas_call(
        _tc1_body,
        grid=(2, N // _BR),
        in_specs=[_rows_spec(), _rows_spec(), _rows_spec()],
        out_specs=_rows_spec(nblk_offset=True),
        out_shape=jax.ShapeDtypeStruct((2 * N, H), _f32),
    )(features, P, nsb)

    agg1 = _make_sc_spmm()(xcat, ecat)

    h1 = pl.pallas_call(
        _tc2_body,
        grid=(2, N // _BR),
        in_specs=[
            _rows_spec(nblk_offset=True),
            _rows_spec(),
            _rows_spec(),
            pl.BlockSpec((H, H), lambda h, r: (0, 0)),
            pl.BlockSpec((8, H), lambda h, r: (0, 0)),
        ],
        out_specs=_rows_spec(nblk_offset=True),
        out_shape=jax.ShapeDtypeStruct((2 * N, H), _f32),
    )(agg1, ndb, nsb, W0, b0r)

    agg2 = _make_sc_spmm()(h1, ecat)

    cs = pl.pallas_call(
        _tca_body,
        grid=(N // _BR,),
        in_specs=[pl.BlockSpec((_BR, H), lambda r: (r, 0)),
                  pl.BlockSpec((_BR, H), lambda r: (r, 0))],
        out_specs=pl.BlockSpec((8, H), lambda r: (0, 0)),
        out_shape=jax.ShapeDtypeStruct((8, H), _f32),
    )(agg2, ndb)

    s2, c0 = pl.pallas_call(
        _tcb_body,
        out_shape=(jax.ShapeDtypeStruct((8, H), _f32),
                   jax.ShapeDtypeStruct((8, 8), _f32)),
    )(cs, W1, b1r, Wd)

    loss = pl.pallas_call(
        _tcc_body,
        grid=(2, N // _BR),
        in_specs=[
            _rows_spec(nblk_offset=True),
            _rows_spec(),
            pl.BlockSpec((8, H), lambda h, r: (0, 0)),
            pl.BlockSpec((8, 8), lambda h, r: (0, 0)),
        ],
        out_specs=pl.BlockSpec((8, 8), lambda h, r: (0, 0)),
        out_shape=jax.ShapeDtypeStruct((8, 8), _f32),
    )(agg2, ndb, s2, c0)

    return loss[0, 0]


# speedup vs baseline: 7.7734x; 1.0017x over previous
import functools

import jax
import jax.numpy as jnp
from jax import lax
from jax.experimental import pallas as pl
from jax.experimental.pallas import tpu as pltpu
from jax.experimental.pallas import tpu_sc as plsc

N = 10000
E = 320000
H = 128
NPAD = 10240
NC = 2
NSC = 16
L = 16

EPT = E // NSC
CH = 80
NFULL = EPT // CH
NB4 = NFULL // 4
LEFT = NFULL - NB4 * 4
ROWS_PT = NPAD // NSC
WCH = 80


def _norm(d):
    return jnp.where(d > 0.0, lax.rsqrt(jnp.maximum(d, 1.0)), 0.0)


@functools.lru_cache(maxsize=None)
def _make_sc_stats():
    mesh = plsc.VectorSubcoreMesh(core_axis_name="c", subcore_axis_name="s")

    @functools.partial(
        pl.kernel,
        out_type=(
            jax.ShapeDtypeStruct((NPAD, H), jnp.float32),
            jax.ShapeDtypeStruct((NPAD, H), jnp.float32),
            jax.ShapeDtypeStruct((N, H), jnp.float32),
        ),
        mesh=mesh,
        compiler_params=pltpu.CompilerParams(needs_layout_passes=False),
        scratch_types=(
            pltpu.VMEM((640,), jnp.float32),
            pltpu.VMEM((CH,), jnp.float32),
            tuple(pltpu.VMEM((CH,), jnp.int32) for _ in range(4)),
            tuple(pltpu.VMEM((CH,), jnp.int32) for _ in range(4)),
            tuple(pltpu.SemaphoreType.DMA for _ in range(4)),
            tuple(pltpu.SemaphoreType.DMA for _ in range(4)),
            pltpu.VMEM((104,), jnp.int32),
            pltpu.VMEM((104, H), jnp.float32),
            pltpu.VMEM((16,), jnp.int32),
            pltpu.VMEM((16, H), jnp.float32),
            pltpu.VMEM((ROWS_PT,), jnp.float32),
            pltpu.VMEM((64, H), jnp.float32),
            pltpu.VMEM_SHARED((NPAD,), jnp.float32),
            pltpu.SemaphoreType.DMA,
        ),
    )
    def sc_stats(ecat, feat, permv, nsb, ndb, pout,
                 zb, ob, ib, ib2, dis, dcs, pidx, prow, pidx16,
                 prow16, dbuf, rowbuf, dacc, sem):
        c = lax.axis_index("c")
        s = lax.axis_index("s")

        for j in range(ROWS_PT // L):
            zb[pl.ds(j * L, L)] = jnp.zeros((L,), jnp.float32)
        for j in range(CH // L):
            ob[pl.ds(j * L, L)] = jnp.ones((L,), jnp.float32)
        pltpu.sync_copy(zb, dacc.at[pl.ds(s * ROWS_PT, ROWS_PT)])
        plsc.subcore_barrier()

        base = c * E + s * EPT
        for u in range(4):
            pltpu.async_copy(ecat.at[pl.ds(base + u * CH, CH)],
                             ib[u], dis[u])

        def dstep(i, _):
            for u in range(4):
                k = 4 * i + u
                off = base + k * CH
                pltpu.make_async_copy(
                    ecat.at[pl.ds(off, CH)], ib[u], dis[u]).wait()

                @pl.when(i > 0)
                def _w():
                    pltpu.make_async_copy(
                        ob, dacc.at[ib2[u]], dcs[u]).wait()
                for j in range(CH // L):
                    ib2[u][pl.ds(j * L, L)] = ib[u][pl.ds(j * L, L)]
                kn = jnp.where(k + 4 < NFULL, k + 4, 0)
                pltpu.async_copy(ecat.at[pl.ds(base + kn * CH, CH)],
                                 ib[u], dis[u])
                pltpu.async_copy(ob, dacc.at[ib2[u]], dcs[u], add=True)
            return ()

        lax.fori_loop(0, NB4, dstep, ())
        for u in range(4):
            pltpu.make_async_copy(
                ecat.at[pl.ds(base, CH)], ib[u], dis[u]).wait()
        for u in range(LEFT):
            pltpu.make_async_copy(ob, dacc.at[ib2[u]], dcs[u]).wait()
            for j in range(CH // L):
                ib2[u][pl.ds(j * L, L)] = ib[u][pl.ds(j * L, L)]
            pltpu.async_copy(ob, dacc.at[ib2[u]], dcs[u], add=True)
        for u in range(4):
            pltpu.make_async_copy(ob, dacc.at[ib2[u]], dcs[u]).wait()

        w = s * NC + c
        for k in range(3):
            pb = w * 312 + k * 104
            pltpu.sync_copy(permv.at[pl.ds(pb, 104)], pidx)
            pltpu.async_copy(feat.at[pidx], prow, sem).wait()
            pltpu.sync_copy(prow, pout.at[pl.ds(pb, 104)])

        @pl.when(w == NC * NSC - 1)
        def _tail():
            pltpu.sync_copy(permv.at[pl.ds(9984, 16)], pidx16)
            pltpu.async_copy(feat.at[pidx16], prow16, sem).wait()
            pltpu.sync_copy(prow16, pout.at[pl.ds(9984, 16)])

        plsc.subcore_barrier()

        pltpu.sync_copy(dacc.at[pl.ds(s * ROWS_PT, ROWS_PT)], dbuf)

        def bcast(out_ref):
            def chunk(ch, _):
                for r in range(64):
                    nv = plsc.load_gather(
                        dbuf, [jnp.full((L,), ch * 64 + r, jnp.int32)])
                    for j in range(H // L):
                        rowbuf[r, pl.ds(j * L, L)] = nv
                pltpu.sync_copy(
                    rowbuf, out_ref.at[pl.ds(s * ROWS_PT + ch * 64, 64)])
                return ()
            lax.fori_loop(0, ROWS_PT // 64, chunk, ())

        @pl.when(c == 0)
        def _w0():
            bcast(nsb)

        @pl.when(c == 1)
        def _w1():
            bcast(ndb)

    return sc_stats


@functools.lru_cache(maxsize=None)
def _make_sc_spmm():
    mesh = plsc.VectorSubcoreMesh(core_axis_name="c", subcore_axis_name="s")

    @functools.partial(
        pl.kernel,
        out_type=jax.ShapeDtypeStruct((2 * N, H), jnp.float32),
        mesh=mesh,
        compiler_params=pltpu.CompilerParams(needs_layout_passes=False),
        scratch_types=(
            tuple(pltpu.VMEM((CH,), jnp.int32) for _ in range(4)),
            tuple(pltpu.VMEM((CH,), jnp.int32) for _ in range(4)),
            tuple(pltpu.VMEM((CH,), jnp.int32) for _ in range(4)),
            tuple(pltpu.VMEM((CH,), jnp.int32) for _ in range(4)),
            tuple(pltpu.VMEM((CH, H), jnp.float32) for _ in range(4)),
            pltpu.VMEM_SHARED((NPAD, H), jnp.float32),
            tuple(pltpu.SemaphoreType.DMA for _ in range(4)),
            tuple(pltpu.SemaphoreType.DMA for _ in range(4)),
            tuple(pltpu.SemaphoreType.DMA for _ in range(4)),
            tuple(pltpu.SemaphoreType.DMA for _ in range(4)),
        ),
    )
    def sc_spmm(tbl, ecat, agg,
                srcb, dstb, srcb2, dstb2, rows, acc,
                sis, sid, sg, sc):
        c = lax.axis_index("c")
        s = lax.axis_index("s")
        roff = c * N

        def zrow(r, _):
            for j in range(H // L):
                rows[0][r, pl.ds(j * L, L)] = jnp.zeros((L,), jnp.float32)
            return ()
        lax.fori_loop(0, WCH, zrow, ())

        def zchunk(k, _):
            pltpu.sync_copy(rows[0],
                            acc.at[pl.ds(s * ROWS_PT + k * WCH, WCH)])
            return ()
        lax.fori_loop(0, ROWS_PT // WCH, zchunk, ())
        plsc.subcore_barrier()

        base = s * EPT

        for u in range(4):
            pltpu.async_copy(ecat.at[pl.ds(base + u * CH, CH)],
                             srcb[u], sis[u])
            pltpu.async_copy(ecat.at[pl.ds(E + base + u * CH, CH)],
                             dstb[u], sid[u])

        def body(i, _):
            for u in range(4):
                k = 4 * i + u
                off = base + k * CH
                pltpu.make_async_copy(
                    ecat.at[pl.ds(off, CH)], srcb[u], sis[u]).wait()
                pltpu.make_async_copy(
                    ecat.at[pl.ds(E + off, CH)], dstb[u], sid[u]).wait()

                @pl.when(i > 0)
                def _w():
                    pltpu.make_async_copy(
                        rows[u], acc.at[dstb2[u]], sc[u]).wait()
                for j in range(CH // L):
                    srcb2[u][pl.ds(j * L, L)] = (
                        srcb[u][pl.ds(j * L, L)] + roff)
                    dstb2[u][pl.ds(j * L, L)] = dstb[u][pl.ds(j * L, L)]
                kn = jnp.where(k + 4 < NFULL, k + 4, 0)
                pltpu.async_copy(ecat.at[pl.ds(base + kn * CH, CH)],
                                 srcb[u], sis[u])
                pltpu.async_copy(ecat.at[pl.ds(E + base + kn * CH, CH)],
                                 dstb[u], sid[u])
                pltpu.async_copy(tbl.at[srcb2[u]], rows[u], sg[u])
            for u in range(4):
                pltpu.make_async_copy(tbl.at[srcb2[u]], rows[u],
                                      sg[u]).wait()
                pltpu.async_copy(rows[u], acc.at[dstb2[u]], sc[u],
                                 add=True)
            return ()

        lax.fori_loop(0, NB4, body, ())
        for u in range(4):
            pltpu.make_async_copy(
                ecat.at[pl.ds(base, CH)], srcb[u], sis[u]).wait()
            pltpu.make_async_copy(
                ecat.at[pl.ds(base, CH)], dstb[u], sid[u]).wait()
        for u in range(LEFT):
            pltpu.make_async_copy(rows[u], acc.at[dstb2[u]], sc[u]).wait()
            for j in range(CH // L):
                srcb2[u][pl.ds(j * L, L)] = srcb[u][pl.ds(j * L, L)] + roff
                dstb2[u][pl.ds(j * L, L)] = dstb[u][pl.ds(j * L, L)]
            pltpu.async_copy(tbl.at[srcb2[u]], rows[u], sg[u])
        for u in range(LEFT):
            pltpu.make_async_copy(tbl.at[srcb2[u]], rows[u], sg[u]).wait()
            pltpu.async_copy(rows[u], acc.at[dstb2[u]], sc[u], add=True)
        for u in range(LEFT, 4):
            pltpu.make_async_copy(rows[u], acc.at[dstb2[u]], sc[u]).wait()
        for u in range(LEFT):
            pltpu.make_async_copy(rows[u], acc.at[dstb2[u]], sc[u]).wait()

        plsc.subcore_barrier()

        nch = jnp.where(s == NSC - 1, (N - (NSC - 1) * ROWS_PT) // WCH,
                        ROWS_PT // WCH)

        def wchunk(k, _):
            r0 = s * ROWS_PT + k * WCH
            pltpu.sync_copy(acc.at[pl.ds(r0, WCH)], rows[0])
            pltpu.sync_copy(rows[0], agg.at[pl.ds(roff + r0, WCH)])
            return ()
        lax.fori_loop(0, nch, wchunk, ())

    return sc_spmm


def _tc1_body(f_ref, p_ref, ns_ref, o_ref):
    h = pl.program_id(0)

    @pl.when(h == 0)
    def _pos():
        o_ref[...] = f_ref[...] * _norm(ns_ref[...])

    @pl.when(h == 1)
    def _neg():
        o_ref[...] = p_ref[...] * _norm(ns_ref[...])


def _tc2_body(a_ref, nd_ref, ns_ref, w_ref, b_ref, o_ref):
    z = a_ref[...] * _norm(nd_ref[...])
    y = jnp.dot(z, w_ref[...], preferred_element_type=jnp.float32)
    y = jnp.maximum(y + b_ref[0:1, :], 0.0)
    o_ref[...] = y * _norm(ns_ref[...])


def _tcr_body(a_ref, nd_ref, w1_ref, b1_ref, wd_ref, o_ref, cs_ref,
              s2_ref):
    i = pl.program_id(0)
    nb = N // _BR

    @pl.when(i == 0)
    def _init():
        cs_ref[...] = jnp.zeros_like(cs_ref)
        o_ref[...] = jnp.zeros_like(o_ref)

    z = a_ref[...] * _norm(nd_ref[...])

    @pl.when(i < nb)
    def _colsum():
        cs_ref[...] += jnp.broadcast_to(
            jnp.sum(z, axis=0, keepdims=True), cs_ref.shape)

    @pl.when(i == nb)
    def _head():
        m = cs_ref[0:1, :] / jnp.float32(N)
        summary = jax.nn.sigmoid(
            jnp.dot(m, w1_ref[...], preferred_element_type=jnp.float32)
            + b1_ref[0:1, :])
        s = lax.dot_general(summary, wd_ref[...], (((1,), (1,)), ((), ())),
                            preferred_element_type=jnp.float32)
        s2 = lax.dot_general(s, w1_ref[...], (((1,), (1,)), ((), ())),
                             preferred_element_type=jnp.float32)
        c0 = jnp.sum(b1_ref[0:1, :] * s)
        s2_ref[0:1, :] = s2
        s2_ref[1:2, :] = jnp.full((1, H), c0, jnp.float32)

    @pl.when(i >= nb)
    def _loss():
        lg = jnp.sum(z * s2_ref[0:1, :], axis=1) + s2_ref[1, 0]
        x = jnp.where(i < 2 * nb, -lg, lg)
        v = jnp.maximum(x, 0.0) + jnp.log(1.0 + jnp.exp(-jnp.abs(x)))
        o_ref[...] += jnp.full(o_ref.shape, jnp.sum(v) / jnp.float32(N))


_BR = 1000
_f32 = jnp.float32


def _rows_spec(nblk_offset=False):
    if nblk_offset:
        return pl.BlockSpec((_BR, H), lambda h, r: (h * (N // _BR) + r, 0))
    return pl.BlockSpec((_BR, H), lambda h, r: (r, 0))


def kernel(features, edge_index, W0, b0, W1, b1, Wd):
    ecat = edge_index.reshape(-1)
    perm = jax.random.permutation(jax.random.key(42), N).astype(jnp.int32)
    b0r = jnp.broadcast_to(b0.reshape(1, H), (8, H))
    b1r = jnp.broadcast_to(b1.reshape(1, H), (8, H))

    nsb, ndb, P = _make_sc_stats()(ecat, features, perm)

    xcat = pl.pallas_call(
        _tc1_body,
        grid=(2, N // _BR),
        in_specs=[_rows_spec(), _rows_spec(), _rows_spec()],
        out_specs=_rows_spec(nblk_offset=True),
        out_shape=jax.ShapeDtypeStruct((2 * N, H), _f32),
    )(features, P, nsb)

    agg1 = _make_sc_spmm()(xcat, ecat)

    h1 = pl.pallas_call(
        _tc2_body,
        grid=(2, N // _BR),
        in_specs=[
            _rows_spec(nblk_offset=True),
            _rows_spec(),
            _rows_spec(),
            pl.BlockSpec((H, H), lambda h, r: (0, 0)),
            pl.BlockSpec((8, H), lambda h, r: (0, 0)),
        ],
        out_specs=_rows_spec(nblk_offset=True),
        out_shape=jax.ShapeDtypeStruct((2 * N, H), _f32),
    )(agg1, ndb, nsb, W0, b0r)

    agg2 = _make_sc_spmm()(h1, ecat)

    nb = N // _BR
    loss = pl.pallas_call(
        _tcr_body,
        grid=(3 * nb,),
        in_specs=[
            pl.BlockSpec((_BR, H),
                         lambda i: (jnp.where(i < nb, i, i - nb), 0)),
            pl.BlockSpec((_BR, H),
                         lambda i: (jnp.where(i < nb, i,
                                              (i - nb) % nb), 0)),
            pl.BlockSpec((H, H), lambda i: (0, 0)),
            pl.BlockSpec((8, H), lambda i: (0, 0)),
            pl.BlockSpec((H, H), lambda i: (0, 0)),
        ],
        out_specs=pl.BlockSpec((8, H), lambda i: (0, 0)),
        out_shape=jax.ShapeDtypeStruct((8, H), _f32),
        scratch_shapes=[pltpu.VMEM((8, H), _f32), pltpu.VMEM((8, H), _f32)],
    )(agg2, ndb, W1, b1r, Wd)

    return loss[0, 0]
